# baseline (speedup 1.0000x reference)
import jax
import jax.numpy as jnp
from jax.experimental import pallas as pl
from jax.experimental.pallas import tpu as pltpu


def _round_up(x, m):
    return ((x + m - 1) // m) * m


def _fused_body(x_ref, w_ref, adj_ref, b_ref, o_ref, xb_ref, wb_ref):
    j = pl.program_id(1)

    @pl.when(j == 0)
    def _():
        xb_ref[...] = x_ref[...].astype(jnp.bfloat16)
        wb_ref[...] = w_ref[...].astype(jnp.bfloat16)

    adj = adj_ref[...].astype(jnp.bfloat16)
    t = jnp.dot(adj, xb_ref[...], preferred_element_type=jnp.float32)
    o_ref[...] = jnp.dot(
        t.astype(jnp.bfloat16), wb_ref[...],
        preferred_element_type=jnp.float32) + b_ref[...]


def kernel(x, adj, weight, bias):
    n, f_in = x.shape
    f_out = weight.shape[1]

    f_in_p = _round_up(f_in, 128)
    f_out_p = _round_up(f_out, 128)

    tm = 1024
    n_p = _round_up(n, 2 * tm)
    steps = n_p // (2 * tm)

    x = x.astype(jnp.float32)
    if (n_p, f_in_p) != (n, f_in):
        x = jnp.pad(x, ((0, n_p - n), (0, f_in_p - f_in)))
    w = weight.astype(jnp.float32)
    if (f_in_p, f_out_p) != (f_in, f_out):
        w = jnp.pad(w, ((0, f_in_p - f_in), (0, f_out_p - f_out)))
    adj_p = adj if n_p == n else jnp.pad(adj, ((0, n_p - n), (0, n_p - n)))
    if bias is None:
        b = jnp.zeros((1, f_out_p), jnp.float32)
    else:
        b = jnp.pad(bias.reshape(1, f_out).astype(jnp.float32),
                    ((0, 0), (0, f_out_p - f_out)))

    out_p = pl.pallas_call(
        _fused_body,
        out_shape=jax.ShapeDtypeStruct((n_p, f_out_p), jnp.float32),
        grid=(2, steps),
        in_specs=[
            pl.BlockSpec((n_p, f_in_p), lambda c, j: (0, 0)),
            pl.BlockSpec((f_in_p, f_out_p), lambda c, j: (0, 0)),
            pl.BlockSpec((tm, n_p),
                         lambda c, j, _s=steps: (c * _s + j, 0)),
            pl.BlockSpec((1, f_out_p), lambda c, j: (0, 0)),
        ],
        out_specs=pl.BlockSpec((tm, f_out_p),
                               lambda c, j, _s=steps: (c * _s + j, 0)),
        scratch_shapes=[
            pltpu.VMEM((n_p, f_in_p), jnp.bfloat16),
            pltpu.VMEM((f_in_p, f_out_p), jnp.bfloat16),
        ],
        compiler_params=pltpu.CompilerParams(
            dimension_semantics=("parallel", "arbitrary"),
            vmem_limit_bytes=48 << 20,
        ),
    )(x, w, adj_p, b)

    return out_p[:n, :f_out]


# speedup vs baseline: 1.0465x; 1.0465x over previous
import jax
import jax.numpy as jnp
from jax.experimental import pallas as pl
from jax.experimental.pallas import tpu as pltpu


def _round_up(x, m):
    return ((x + m - 1) // m) * m


def _fused_body(x_ref, w_ref, adj_ref, b_ref, o_ref, xb_ref, wb_ref):
    j = pl.program_id(0)

    @pl.when(j == 0)
    def _():
        xb_ref[...] = x_ref[...].astype(jnp.bfloat16)
        wb_ref[...] = w_ref[...].astype(jnp.bfloat16)

    adj = adj_ref[...].astype(jnp.bfloat16)
    t = jnp.dot(adj, xb_ref[...], preferred_element_type=jnp.float32)
    o_ref[...] = jnp.dot(
        t.astype(jnp.bfloat16), wb_ref[...],
        preferred_element_type=jnp.float32) + b_ref[...]


def kernel(x, adj, weight, bias):
    n, f_in = x.shape
    f_out = weight.shape[1]

    f_in_p = _round_up(f_in, 128)
    f_out_p = _round_up(f_out, 128)

    tm = 512
    n_p = _round_up(n, tm)
    steps = n_p // tm

    x = x.astype(jnp.float32)
    if (n_p, f_in_p) != (n, f_in):
        x = jnp.pad(x, ((0, n_p - n), (0, f_in_p - f_in)))
    w = weight.astype(jnp.float32)
    if (f_in_p, f_out_p) != (f_in, f_out):
        w = jnp.pad(w, ((0, f_in_p - f_in), (0, f_out_p - f_out)))
    adj_p = adj if n_p == n else jnp.pad(adj, ((0, n_p - n), (0, n_p - n)))
    if bias is None:
        b = jnp.zeros((1, f_out_p), jnp.float32)
    else:
        b = jnp.pad(bias.reshape(1, f_out).astype(jnp.float32),
                    ((0, 0), (0, f_out_p - f_out)))

    out_p = pl.pallas_call(
        _fused_body,
        out_shape=jax.ShapeDtypeStruct((n_p, f_out_p), jnp.float32),
        grid=(steps,),
        in_specs=[
            pl.BlockSpec((n_p, f_in_p), lambda j: (0, 0)),
            pl.BlockSpec((f_in_p, f_out_p), lambda j: (0, 0)),
            pl.BlockSpec((tm, n_p), lambda j: (j, 0)),
            pl.BlockSpec((1, f_out_p), lambda j: (0, 0)),
        ],
        out_specs=pl.BlockSpec((tm, f_out_p), lambda j: (j, 0)),
        scratch_shapes=[
            pltpu.VMEM((n_p, f_in_p), jnp.bfloat16),
            pltpu.VMEM((f_in_p, f_out_p), jnp.bfloat16),
        ],
        compiler_params=pltpu.CompilerParams(
            dimension_semantics=("arbitrary",),
            vmem_limit_bytes=48 << 20,
        ),
    )(x, w, adj_p, b)

    return out_p[:n, :f_out]
